# baseline (speedup 1.0000x reference)
import functools

import jax
import jax.numpy as jnp
from jax import lax
from jax.experimental import pallas as pl
from jax.experimental.pallas import tpu as pltpu
from jax.experimental.pallas import tpu_sc as plsc

_N = 10000
_E = 320000
_H = 128

_EBLK = 512
_CH = 128
_NCHUNK = _E // _CH
_NCPAD = 2504
_NTILES = 32
_NPAD = 10240
_ZR = 128

_HC0 = 1280
_HC1 = _NCHUNK - _HC0
_GPT = 40
_SPT = 80
_PREM = 24

_mesh = plsc.VectorSubcoreMesh(core_axis_name="c", subcore_axis_name="s")


def _bf16_bits(x):
    b = jax.lax.bitcast_convert_type(x.astype(jnp.bfloat16), jnp.uint16)
    return b.astype(jnp.int32)


def _pack2(lo, hi):
    return (_bf16_bits(hi) << 16) | _bf16_bits(lo)


def _zero_vmem(buf):
    z = jnp.zeros((16,), jnp.float32)

    @pl.loop(0, buf.shape[0])
    def _(r):
        @pl.loop(0, buf.shape[1], step=16)
        def _(c):
            buf[r, pl.ds(c, 16)] = z


def _preload_idx(src2d_hbm, idx, r0, count, full):
    @pl.when(count >= full)
    def _():
        pltpu.sync_copy(src2d_hbm.at[pl.ds(r0, full)], idx)

    @pl.when(jnp.logical_and(count > 0, count < full))
    def _():
        pltpu.sync_copy(src2d_hbm.at[pl.ds(r0, _PREM)],
                        idx.at[pl.ds(0, _PREM)])


def _make_gather(nch):
    def body(tsrc_hbm, tdst_hbm, src2d_hbm, dst2d_hbm, g_hbm, gd_hbm,
             idx_s, idx_d, bs0, bs1, bd0, bd1,
             sgs0, sgs1, sgd0, sgd1, sw0, sw1, swd0, swd1):
        cid = lax.axis_index("c")
        sid = lax.axis_index("s")
        wid = sid * 2 + cid
        r0 = wid * _GPT
        count = jnp.maximum(0, jnp.minimum(_GPT, nch - r0))

        _preload_idx(src2d_hbm, idx_s, r0, count, _GPT)
        _preload_idx(dst2d_hbm, idx_d, r0, count, _GPT)

        def issue_g(i, bs, bd, sg_s, sg_d):
            pltpu.async_copy(tsrc_hbm.at[idx_s.at[i]], bs, sg_s)
            pltpu.async_copy(tdst_hbm.at[idx_d.at[i]], bd, sg_d)

        def wait_g(bs, bd, sg_s, sg_d):
            pltpu.make_async_copy(tsrc_hbm.at[idx_s.at[0]], bs, sg_s).wait()
            pltpu.make_async_copy(tdst_hbm.at[idx_d.at[0]], bd, sg_d).wait()

        def issue_w(i, bs, bd, sw, swd):
            base = (r0 + i) * _CH
            pltpu.async_copy(bs, g_hbm.at[pl.ds(base, _CH)], sw)
            pltpu.async_copy(bd, gd_hbm.at[pl.ds(base, _CH)], swd)

        def wait_w(bs, bd, sw, swd):
            pltpu.make_async_copy(bs, g_hbm.at[pl.ds(0, _CH)], sw).wait()
            pltpu.make_async_copy(bd, gd_hbm.at[pl.ds(0, _CH)], swd).wait()

        @pl.when(count > 0)
        def _():
            issue_g(0, bs0, bd0, sgs0, sgd0)

        def step(j, carry):
            a = 2 * j

            wait_g(bs0, bd0, sgs0, sgd0)

            @pl.when(a + 1 < count)
            def _():
                @pl.when(j > 0)
                def _():
                    wait_w(bs1, bd1, sw1, swd1)
                issue_g(a + 1, bs1, bd1, sgs1, sgd1)

            issue_w(a, bs0, bd0, sw0, swd0)

            @pl.when(a + 1 < count)
            def _():
                wait_g(bs1, bd1, sgs1, sgd1)

                @pl.when(a + 2 < count)
                def _():
                    wait_w(bs0, bd0, sw0, swd0)
                    issue_g(a + 2, bs0, bd0, sgs0, sgd0)

                issue_w(a + 1, bs1, bd1, sw1, swd1)

            return carry

        lax.fori_loop(0, (count + 1) // 2, step, 0)

        @pl.when(count > 0)
        def _():
            wait_w(bs0, bd0, sw0, swd0)

        @pl.when(count > 1)
        def _():
            wait_w(bs1, bd1, sw1, swd1)

    ne = nch * _CH

    @jax.jit
    def call(table_src, table_dst, src2d, dst2d):
        return pl.kernel(
            body,
            out_type=[jax.ShapeDtypeStruct((ne, _H), jnp.int32),
                      jax.ShapeDtypeStruct((ne, _H), jnp.float32)],
            mesh=_mesh,
            scratch_types=[
                pltpu.VMEM((_GPT, _CH), jnp.int32),
                pltpu.VMEM((_GPT, _CH), jnp.int32),
                pltpu.VMEM((_CH, _H), jnp.int32),
                pltpu.VMEM((_CH, _H), jnp.int32),
                pltpu.VMEM((_CH, _H), jnp.float32),
                pltpu.VMEM((_CH, _H), jnp.float32),
            ] + [pltpu.SemaphoreType.DMA] * 8,
        )(table_src, table_dst, src2d, dst2d)

    return call


def _make_scatter(nch):
    def sloop(d_hbm, dst2d_hbm, acc, idx, c0, c1, sl0, sl1, sid):
        r0 = sid * _SPT
        count = jnp.maximum(0, jnp.minimum(_SPT, nch - r0))

        _preload_idx(dst2d_hbm, idx, r0, count, _SPT)

        def issue_l(i, cb, sl):
            base = (r0 + i) * _CH
            pltpu.async_copy(d_hbm.at[pl.ds(base, _CH)], cb, sl)

        def wait_l(cb, sl):
            pltpu.make_async_copy(d_hbm.at[pl.ds(0, _CH)], cb, sl).wait()

        @pl.when(count > 0)
        def _():
            issue_l(0, c0, sl0)

        def step(j, carry):
            a = 2 * j

            wait_l(c0, sl0)

            @pl.when(a + 1 < count)
            def _():
                issue_l(a + 1, c1, sl1)

            pltpu.sync_copy(c0, acc.at[idx.at[a]], add=True)

            @pl.when(a + 1 < count)
            def _():
                wait_l(c1, sl1)

                @pl.when(a + 2 < count)
                def _():
                    issue_l(a + 2, c0, sl0)

                pltpu.sync_copy(c1, acc.at[idx.at[a + 1]], add=True)

            return carry

        lax.fori_loop(0, (count + 1) // 2, step, 0)

    def body(msg_hbm, sig_hbm, dst2d_hbm, num_hbm, den_hbm,
             acc, idx, c0, c1, sl0, sl1):
        cid = lax.axis_index("c")
        sid = lax.axis_index("s")

        _zero_vmem(c0)
        row0 = sid * (_NPAD // 16)

        @pl.loop(0, _NPAD // 16, step=_ZR)
        def _(k):
            pltpu.sync_copy(c0, acc.at[pl.ds(row0 + k, _ZR)])

        plsc.subcore_barrier()

        @pl.when(cid == 0)
        def _():
            sloop(msg_hbm, dst2d_hbm, acc, idx, c0, c1, sl0, sl1, sid)

        @pl.when(cid == 1)
        def _():
            sloop(sig_hbm, dst2d_hbm, acc, idx, c0, c1, sl0, sl1, sid)

        plsc.subcore_barrier()

        @pl.when(cid == 0)
        def _():
            @pl.loop(0, _NPAD // 16, step=_ZR)
            def _(k):
                pltpu.sync_copy(acc.at[pl.ds(row0 + k, _ZR)],
                                num_hbm.at[pl.ds(row0 + k, _ZR)])

        @pl.when(cid == 1)
        def _():
            @pl.loop(0, _NPAD // 16, step=_ZR)
            def _(k):
                pltpu.sync_copy(acc.at[pl.ds(row0 + k, _ZR)],
                                den_hbm.at[pl.ds(row0 + k, _ZR)])

    @jax.jit
    def call(msg, sig, dst2d):
        return pl.kernel(
            body,
            out_type=[jax.ShapeDtypeStruct((_NPAD, _H), jnp.float32),
                      jax.ShapeDtypeStruct((_NPAD, _H), jnp.float32)],
            mesh=_mesh,
            scratch_types=[
                pltpu.VMEM_SHARED((_NPAD, _H), jnp.float32),
                pltpu.VMEM((_SPT, _CH), jnp.int32),
                pltpu.VMEM((_CH, _H), jnp.float32),
                pltpu.VMEM((_CH, _H), jnp.float32),
                pltpu.SemaphoreType.DMA,
                pltpu.SemaphoreType.DMA,
            ],
        )(msg, sig, dst2d)

    return call


def _edgewise_body(g_ref, gd_ref, c_ref, ein_ref, sn_ref,
                   enew_ref, sig_ref, msg_ref):
    gw = g_ref[...]
    eh_src = jax.lax.bitcast_convert_type(gw << 16, jnp.float32)
    bh_src = jax.lax.bitcast_convert_type(
        gw & jnp.int32(-65536), jnp.float32)
    dh_dst = gd_ref[...]
    ce = jnp.dot(ein_ref[...].astype(jnp.bfloat16),
                 c_ref[...].astype(jnp.bfloat16),
                 preferred_element_type=jnp.float32)
    e_hat = dh_dst + eh_src + ce
    sig = jax.nn.sigmoid(e_hat)
    sig_ref[...] = sig
    msg_ref[...] = sig * bh_src
    enew_ref[...] = ein_ref[...] + jax.nn.relu(e_hat * sn_ref[...])


def _edgewise(g, gd, C, e_in, snorm_e):
    ne = g.shape[0]
    n_blk = ne // _EBLK
    spec = pl.BlockSpec((_EBLK, _H), lambda i: (i, 0))
    spec1 = pl.BlockSpec((_EBLK, 1), lambda i: (i, 0))
    specw = pl.BlockSpec((_H, _H), lambda i: (0, 0))
    out_shapes = [jax.ShapeDtypeStruct((ne, _H), jnp.float32)] * 3
    return pl.pallas_call(
        _edgewise_body,
        grid=(n_blk,),
        in_specs=[spec, spec, specw, spec, spec1],
        out_specs=[spec, spec, spec],
        out_shape=out_shapes,
    )(g, gd, C, e_in, snorm_e)


_gather_h = (_make_gather(_HC0), _make_gather(_HC1))
_scatter_h = (_make_scatter(_HC0), _make_scatter(_HC1))
_NE0 = _HC0 * _CH
_NE1 = _HC1 * _CH


def kernel(h, e, snorm_n, snorm_e, W_emb_h, W_emb_e, W_layers, W_ro, W_pred,
           b_pred, edge_index):
    pad = _NCPAD * _CH - _E
    src2d = jnp.pad(edge_index[0], (0, pad)).reshape(_NCPAD, _CH)
    dst2d = jnp.pad(edge_index[1], (0, pad)).reshape(_NCPAD, _CH)
    src2d_h = (src2d[:_HC0], src2d[_HC0:])
    dst2d_h = (dst2d[:_HC0], dst2d[_HC0:])
    sn_h = (snorm_e[:_NE0], snorm_e[_NE0:])
    h = h @ W_emb_h
    e_h = [e[:_NE0] @ W_emb_e, e[_NE0:] @ W_emb_e]
    for l in range(4):
        h_in = h
        A, B, C, Dw, Ew = (W_layers[l, i] for i in range(5))
        Ah = h @ A
        Bh = h @ B
        Dh = h @ Dw
        Eh = h @ Ew
        table_src = _pack2(Eh, Bh)
        g0, gd0 = _gather_h[0](table_src, Dh, src2d_h[0], dst2d_h[0])
        g1, gd1 = _gather_h[1](table_src, Dh, src2d_h[1], dst2d_h[1])
        e_new0, sig0, msg0 = _edgewise(g0, gd0, C, e_h[0], sn_h[0])
        pn0, pd0 = _scatter_h[0](msg0, sig0, dst2d_h[0])
        e_new1, sig1, msg1 = _edgewise(g1, gd1, C, e_h[1], sn_h[1])
        pn1, pd1 = _scatter_h[1](msg1, sig1, dst2d_h[1])
        e_new_h = [e_new0, e_new1]
        num = pn0 + pn1
        den = pd0 + pd1
        num = num[:_N]
        den = den[:_N]
        h_new = jax.nn.relu((Ah + num / (den + 1e-6)) * snorm_n)
        e_h = e_new_h
        h = h_in + h_new
    hro = h @ W_ro
    hg = jnp.sum(hro, axis=0, keepdims=True)
    return hg @ W_pred + b_pred


# speedup vs baseline: 1.2038x; 1.2038x over previous
import functools

import jax
import jax.numpy as jnp
from jax import lax
from jax.experimental import pallas as pl
from jax.experimental.pallas import tpu as pltpu
from jax.experimental.pallas import tpu_sc as plsc

_N = 10000
_E = 320000
_H = 128

_EBLK = 1280
_CH = 128
_NCHUNK = _E // _CH
_NCPAD = 2504
_NTILES = 32
_NPAD = 10240
_ZR = 128

_HC0 = 1280
_HC1 = _NCHUNK - _HC0
_GPT = 40
_SPT = 80
_PREM = 24

_mesh = plsc.VectorSubcoreMesh(core_axis_name="c", subcore_axis_name="s")


def _bf16_bits(x):
    b = jax.lax.bitcast_convert_type(x.astype(jnp.bfloat16), jnp.uint16)
    return b.astype(jnp.int32)


def _pack2(lo, hi):
    return (_bf16_bits(hi) << 16) | _bf16_bits(lo)


def _zero_vmem(buf):
    z = jnp.zeros((16,), jnp.float32)

    @pl.loop(0, buf.shape[0])
    def _(r):
        @pl.loop(0, buf.shape[1], step=16)
        def _(c):
            buf[r, pl.ds(c, 16)] = z


def _preload_idx(src2d_hbm, idx, r0, count, full):
    @pl.when(count >= full)
    def _():
        pltpu.sync_copy(src2d_hbm.at[pl.ds(r0, full)], idx)

    @pl.when(jnp.logical_and(count > 0, count < full))
    def _():
        pltpu.sync_copy(src2d_hbm.at[pl.ds(r0, _PREM)],
                        idx.at[pl.ds(0, _PREM)])


def _make_gather(nch):
    def body(tsrc_hbm, tdst_hbm, src2d_hbm, dst2d_hbm, g_hbm, gd_hbm,
             idx_s, idx_d, bs0, bs1, bd0, bd1,
             sgs0, sgs1, sgd0, sgd1, sw0, sw1, swd0, swd1):
        cid = lax.axis_index("c")
        sid = lax.axis_index("s")
        wid = sid * 2 + cid
        r0 = wid * _GPT
        count = jnp.maximum(0, jnp.minimum(_GPT, nch - r0))

        _preload_idx(src2d_hbm, idx_s, r0, count, _GPT)
        _preload_idx(dst2d_hbm, idx_d, r0, count, _GPT)

        def issue_g(i, bs, bd, sg_s, sg_d):
            pltpu.async_copy(tsrc_hbm.at[idx_s.at[i]], bs, sg_s)
            pltpu.async_copy(tdst_hbm.at[idx_d.at[i]], bd, sg_d)

        def wait_g(bs, bd, sg_s, sg_d):
            pltpu.make_async_copy(tsrc_hbm.at[idx_s.at[0]], bs, sg_s).wait()
            pltpu.make_async_copy(tdst_hbm.at[idx_d.at[0]], bd, sg_d).wait()

        def issue_w(i, bs, bd, sw, swd):
            base = (r0 + i) * _CH
            pltpu.async_copy(bs, g_hbm.at[pl.ds(base, _CH)], sw)
            pltpu.async_copy(bd, gd_hbm.at[pl.ds(base, _CH)], swd)

        def wait_w(bs, bd, sw, swd):
            pltpu.make_async_copy(bs, g_hbm.at[pl.ds(0, _CH)], sw).wait()
            pltpu.make_async_copy(bd, gd_hbm.at[pl.ds(0, _CH)], swd).wait()

        @pl.when(count > 0)
        def _():
            issue_g(0, bs0, bd0, sgs0, sgd0)

        def step(j, carry):
            a = 2 * j

            wait_g(bs0, bd0, sgs0, sgd0)

            @pl.when(a + 1 < count)
            def _():
                @pl.when(j > 0)
                def _():
                    wait_w(bs1, bd1, sw1, swd1)
                issue_g(a + 1, bs1, bd1, sgs1, sgd1)

            issue_w(a, bs0, bd0, sw0, swd0)

            @pl.when(a + 1 < count)
            def _():
                wait_g(bs1, bd1, sgs1, sgd1)

                @pl.when(a + 2 < count)
                def _():
                    wait_w(bs0, bd0, sw0, swd0)
                    issue_g(a + 2, bs0, bd0, sgs0, sgd0)

                issue_w(a + 1, bs1, bd1, sw1, swd1)

            return carry

        lax.fori_loop(0, (count + 1) // 2, step, 0)

        @pl.when(count > 0)
        def _():
            wait_w(bs0, bd0, sw0, swd0)

        @pl.when(count > 1)
        def _():
            wait_w(bs1, bd1, sw1, swd1)

    ne = nch * _CH

    @jax.jit
    def call(table_src, table_dst, src2d, dst2d):
        return pl.kernel(
            body,
            out_type=[jax.ShapeDtypeStruct((ne, _H), jnp.int32),
                      jax.ShapeDtypeStruct((ne, _H), jnp.float32)],
            mesh=_mesh,
            scratch_types=[
                pltpu.VMEM((_GPT, _CH), jnp.int32),
                pltpu.VMEM((_GPT, _CH), jnp.int32),
                pltpu.VMEM((_CH, _H), jnp.int32),
                pltpu.VMEM((_CH, _H), jnp.int32),
                pltpu.VMEM((_CH, _H), jnp.float32),
                pltpu.VMEM((_CH, _H), jnp.float32),
            ] + [pltpu.SemaphoreType.DMA] * 8,
        )(table_src, table_dst, src2d, dst2d)

    return call


def _make_scatter(nch):
    def sloop(d_hbm, dst2d_hbm, acc, idx, c0, c1, sl0, sl1, sid):
        r0 = sid * _SPT
        count = jnp.maximum(0, jnp.minimum(_SPT, nch - r0))

        _preload_idx(dst2d_hbm, idx, r0, count, _SPT)

        def issue_l(i, cb, sl):
            base = (r0 + i) * _CH
            pltpu.async_copy(d_hbm.at[pl.ds(base, _CH)], cb, sl)

        def wait_l(cb, sl):
            pltpu.make_async_copy(d_hbm.at[pl.ds(0, _CH)], cb, sl).wait()

        @pl.when(count > 0)
        def _():
            issue_l(0, c0, sl0)

        def step(j, carry):
            a = 2 * j

            wait_l(c0, sl0)

            @pl.when(a + 1 < count)
            def _():
                issue_l(a + 1, c1, sl1)

            pltpu.sync_copy(c0, acc.at[idx.at[a]], add=True)

            @pl.when(a + 1 < count)
            def _():
                wait_l(c1, sl1)

                @pl.when(a + 2 < count)
                def _():
                    issue_l(a + 2, c0, sl0)

                pltpu.sync_copy(c1, acc.at[idx.at[a + 1]], add=True)

            return carry

        lax.fori_loop(0, (count + 1) // 2, step, 0)

    def body(msg_hbm, sig_hbm, dst2d_hbm, num_hbm, den_hbm,
             acc, idx, c0, c1, sl0, sl1):
        cid = lax.axis_index("c")
        sid = lax.axis_index("s")

        _zero_vmem(c0)
        row0 = sid * (_NPAD // 16)

        @pl.loop(0, _NPAD // 16, step=_ZR)
        def _(k):
            pltpu.sync_copy(c0, acc.at[pl.ds(row0 + k, _ZR)])

        plsc.subcore_barrier()

        @pl.when(cid == 0)
        def _():
            sloop(msg_hbm, dst2d_hbm, acc, idx, c0, c1, sl0, sl1, sid)

        @pl.when(cid == 1)
        def _():
            sloop(sig_hbm, dst2d_hbm, acc, idx, c0, c1, sl0, sl1, sid)

        plsc.subcore_barrier()

        @pl.when(cid == 0)
        def _():
            @pl.loop(0, _NPAD // 16, step=_ZR)
            def _(k):
                pltpu.sync_copy(acc.at[pl.ds(row0 + k, _ZR)],
                                num_hbm.at[pl.ds(row0 + k, _ZR)])

        @pl.when(cid == 1)
        def _():
            @pl.loop(0, _NPAD // 16, step=_ZR)
            def _(k):
                pltpu.sync_copy(acc.at[pl.ds(row0 + k, _ZR)],
                                den_hbm.at[pl.ds(row0 + k, _ZR)])

    @jax.jit
    def call(msg, sig, dst2d):
        return pl.kernel(
            body,
            out_type=[jax.ShapeDtypeStruct((_NPAD, _H), jnp.float32),
                      jax.ShapeDtypeStruct((_NPAD, _H), jnp.float32)],
            mesh=_mesh,
            scratch_types=[
                pltpu.VMEM_SHARED((_NPAD, _H), jnp.float32),
                pltpu.VMEM((_SPT, _CH), jnp.int32),
                pltpu.VMEM((_CH, _H), jnp.float32),
                pltpu.VMEM((_CH, _H), jnp.float32),
                pltpu.SemaphoreType.DMA,
                pltpu.SemaphoreType.DMA,
            ],
        )(msg, sig, dst2d)

    return call


def _edgewise_body(g_ref, gd_ref, c_ref, ein_ref, sn_ref,
                   enew_ref, sig_ref, msg_ref):
    gw = g_ref[...]
    eh_src = jax.lax.bitcast_convert_type(gw << 16, jnp.float32)
    bh_src = jax.lax.bitcast_convert_type(
        gw & jnp.int32(-65536), jnp.float32)
    dh_dst = gd_ref[...]
    ce = jnp.dot(ein_ref[...].astype(jnp.bfloat16),
                 c_ref[...].astype(jnp.bfloat16),
                 preferred_element_type=jnp.float32)
    e_hat = dh_dst + eh_src + ce
    sig = 0.5 * jnp.tanh(0.5 * e_hat) + 0.5
    sig_ref[...] = sig
    msg_ref[...] = sig * bh_src
    enew_ref[...] = ein_ref[...] + jax.nn.relu(e_hat * sn_ref[...])


def _edgewise(g, gd, C, e_in, snorm_e):
    ne = g.shape[0]
    n_blk = ne // _EBLK
    spec = pl.BlockSpec((_EBLK, _H), lambda i: (i, 0))
    spec1 = pl.BlockSpec((_EBLK, 1), lambda i: (i, 0))
    specw = pl.BlockSpec((_H, _H), lambda i: (0, 0))
    out_shapes = [jax.ShapeDtypeStruct((ne, _H), jnp.float32)] * 3
    return pl.pallas_call(
        _edgewise_body,
        grid=(n_blk,),
        in_specs=[spec, spec, specw, spec, spec1],
        out_specs=[spec, spec, spec],
        out_shape=out_shapes,
    )(g, gd, C, e_in, snorm_e)


_gather_h = (_make_gather(_HC0), _make_gather(_HC1))
_scatter_h = (_make_scatter(_HC0), _make_scatter(_HC1))
_NE0 = _HC0 * _CH
_NE1 = _HC1 * _CH


def kernel(h, e, snorm_n, snorm_e, W_emb_h, W_emb_e, W_layers, W_ro, W_pred,
           b_pred, edge_index):
    pad = _NCPAD * _CH - _E
    src2d = jnp.pad(edge_index[0], (0, pad)).reshape(_NCPAD, _CH)
    dst2d = jnp.pad(edge_index[1], (0, pad)).reshape(_NCPAD, _CH)
    src2d_h = (src2d[:_HC0], src2d[_HC0:])
    dst2d_h = (dst2d[:_HC0], dst2d[_HC0:])
    sn_h = (snorm_e[:_NE0], snorm_e[_NE0:])
    h = h @ W_emb_h
    e_h = [e[:_NE0] @ W_emb_e, e[_NE0:] @ W_emb_e]
    for l in range(4):
        h_in = h
        A, B, C, Dw, Ew = (W_layers[l, i] for i in range(5))
        Ah = h @ A
        Bh = h @ B
        Dh = h @ Dw
        Eh = h @ Ew
        table_src = _pack2(Eh, Bh)
        g0, gd0 = _gather_h[0](table_src, Dh, src2d_h[0], dst2d_h[0])
        g1, gd1 = _gather_h[1](table_src, Dh, src2d_h[1], dst2d_h[1])
        e_new0, sig0, msg0 = _edgewise(g0, gd0, C, e_h[0], sn_h[0])
        pn0, pd0 = _scatter_h[0](msg0, sig0, dst2d_h[0])
        e_new1, sig1, msg1 = _edgewise(g1, gd1, C, e_h[1], sn_h[1])
        pn1, pd1 = _scatter_h[1](msg1, sig1, dst2d_h[1])
        e_new_h = [e_new0, e_new1]
        num = pn0 + pn1
        den = pd0 + pd1
        num = num[:_N]
        den = den[:_N]
        h_new = jax.nn.relu((Ah + num / (den + 1e-6)) * snorm_n)
        e_h = e_new_h
        h = h_in + h_new
    hro = h @ W_ro
    hg = jnp.sum(hro, axis=0, keepdims=True)
    return hg @ W_pred + b_pred
